# baseline (speedup 1.0000x reference)
import functools

import jax
import jax.numpy as jnp
from jax import lax
from jax.experimental import pallas as pl
from jax.experimental.pallas import tpu as pltpu
from jax.experimental.pallas import tpu_sc as plsc

_NC = 1
_NS = 16
_NW = _NC * _NS
_L = 16


@functools.lru_cache(maxsize=None)
def _build(B: int, E: int):
    e_per = E // _NW
    n_vec = e_per // _L
    mesh = plsc.VectorSubcoreMesh(core_axis_name="c", subcore_axis_name="s", num_cores=1)

    @functools.partial(
        pl.kernel,
        mesh=mesh,
        out_type=jax.ShapeDtypeStruct((E,), jnp.int32),
        compiler_params=pltpu.CompilerParams(needs_layout_passes=False),
        scratch_types=[
            pltpu.VMEM((_L,), jnp.int32),
            pltpu.VMEM((_L,), jnp.int32),
            pltpu.VMEM((_L,), jnp.int32),
            pltpu.VMEM((_L,), jnp.int32),
            pltpu.VMEM((e_per,), jnp.int32),
            pltpu.SemaphoreType.DMA,
            pltpu.SemaphoreType.DMA,
            pltpu.SemaphoreType.DMA,
        ],
    )
    def merged_source_kernel(nrl_hbm, erl_hbm, esrc_hbm, out_hbm,
                             nrl_v, erl_v, noff_v, estart_v, src_v,
                             sem_src, sem_n, sem_e):
        wid = lax.axis_index("s") * _NC + lax.axis_index("c")
        base = wid * e_per
        cp_src = pltpu.async_copy(esrc_hbm.at[pl.ds(base, e_per)], src_v,
                                  sem_src)
        cp_n = pltpu.async_copy(nrl_hbm, nrl_v.at[pl.ds(0, B)], sem_n)
        cp_e = pltpu.async_copy(erl_hbm, erl_v.at[pl.ds(0, B)], sem_e)
        cp_n.wait()
        cp_e.wait()

        nrl = nrl_v[...]
        erl = erl_v[...]
        noff_v[...] = jnp.cumsum(nrl) - nrl
        estart_v[...] = jnp.cumsum(erl) - erl

        starts = [
            plsc.load_gather(estart_v, [jnp.full((_L,), j, jnp.int32)])
            for j in range(1, B)
        ]
        cp_src.wait()

        pos0 = base + lax.iota(jnp.int32, _L)

        @plsc.parallel_loop(0, e_per, step=_L, unroll=4)
        def _body(i):
            pos = pos0 + i
            gid = jnp.zeros((_L,), jnp.int32)
            for j, s in enumerate(starts):
                gid = jnp.where(pos >= s, jnp.int32(j + 1), gid)
            off = plsc.load_gather(noff_v, [gid])
            sl = pl.ds(i, _L)
            src_v[sl] = src_v[sl] + off

        pltpu.sync_copy(src_v, out_hbm.at[pl.ds(base, e_per)])

    return merged_source_kernel


def kernel(node_features, node_row_lengths, edge_source, edge_target,
           edge_row_lengths):
    B = node_row_lengths.shape[0]
    E = edge_source.shape[0]
    return _build(B, E)(node_row_lengths, edge_row_lengths, edge_source)


# speedup vs baseline: 1.0044x; 1.0044x over previous
import functools

import jax
import jax.numpy as jnp
from jax import lax
from jax.experimental import pallas as pl
from jax.experimental.pallas import tpu as pltpu
from jax.experimental.pallas import tpu_sc as plsc

_NC = 1
_NS = 16
_NW = _NC * _NS
_L = 16


@functools.lru_cache(maxsize=None)
def _build(B: int, E: int):
    e_per = E // _NW
    mesh = plsc.VectorSubcoreMesh(core_axis_name="c", subcore_axis_name="s",
                                  num_cores=_NC)

    @functools.partial(
        pl.kernel,
        mesh=mesh,
        out_type=jax.ShapeDtypeStruct((E,), jnp.int32),
        compiler_params=pltpu.CompilerParams(needs_layout_passes=False),
        scratch_types=[
            pltpu.VMEM((_L,), jnp.int32),
            pltpu.VMEM((_L,), jnp.int32),
            pltpu.VMEM((_L,), jnp.int32),
            pltpu.VMEM((_L,), jnp.int32),
            pltpu.VMEM((e_per,), jnp.int32),
            pltpu.SemaphoreType.DMA,
            pltpu.SemaphoreType.DMA,
            pltpu.SemaphoreType.DMA,
        ],
    )
    def merged_source_kernel(nrl_hbm, erl_hbm, esrc_hbm, out_hbm,
                             nrl_v, erl_v, noff_v, estart_v, src_v,
                             sem_src, sem_n, sem_e):
        wid = lax.axis_index("s") * _NC + lax.axis_index("c")
        base = wid * e_per
        cp_src = pltpu.async_copy(esrc_hbm.at[pl.ds(base, e_per)], src_v,
                                  sem_src)
        cp_n = pltpu.async_copy(nrl_hbm, nrl_v.at[pl.ds(0, B)], sem_n)
        cp_e = pltpu.async_copy(erl_hbm, erl_v.at[pl.ds(0, B)], sem_e)
        cp_n.wait()
        cp_e.wait()

        nrl = nrl_v[...]
        erl = erl_v[...]
        noff_v[...] = jnp.cumsum(nrl) - nrl
        estart_v[...] = jnp.cumsum(erl) - erl

        starts = [
            plsc.load_gather(estart_v, [jnp.full((_L,), j, jnp.int32)])
            for j in range(1, B)
        ]

        def graph_id(pos):
            gid = jnp.zeros((_L,), jnp.int32)
            for j, s in enumerate(starts):
                gid = jnp.where(pos >= s, jnp.int32(j + 1), gid)
            return gid

        gid_lo = graph_id(jnp.full((_L,), base, jnp.int32))
        gid_hi = graph_id(jnp.full((_L,), base + (e_per - 1), jnp.int32))
        uniform = jnp.min(gid_lo) == jnp.max(gid_hi)
        cp_src.wait()

        @pl.when(uniform)
        def _fast():
            off = plsc.load_gather(noff_v, [gid_lo])

            @plsc.parallel_loop(0, e_per, step=_L, unroll=8)
            def _body(i):
                sl = pl.ds(i, _L)
                src_v[sl] = src_v[sl] + off

        @pl.when(jnp.logical_not(uniform))
        def _general():
            pos0 = base + lax.iota(jnp.int32, _L)

            @plsc.parallel_loop(0, e_per, step=_L, unroll=4)
            def _body(i):
                off = plsc.load_gather(noff_v, [graph_id(pos0 + i)])
                sl = pl.ds(i, _L)
                src_v[sl] = src_v[sl] + off

        pltpu.sync_copy(src_v, out_hbm.at[pl.ds(base, e_per)])

    return merged_source_kernel


def kernel(node_features, node_row_lengths, edge_source, edge_target,
           edge_row_lengths):
    B = node_row_lengths.shape[0]
    E = edge_source.shape[0]
    return _build(B, E)(node_row_lengths, edge_row_lengths, edge_source)


# speedup vs baseline: 1.0081x; 1.0037x over previous
import functools

import jax
import jax.numpy as jnp
from jax import lax
from jax.experimental import pallas as pl
from jax.experimental.pallas import tpu as pltpu
from jax.experimental.pallas import tpu_sc as plsc

_NC = 1
_NS = 16
_NW = _NC * _NS
_L = 16


@functools.lru_cache(maxsize=None)
def _build(B: int, E: int):
    e_per = E // _NW
    mesh = plsc.VectorSubcoreMesh(core_axis_name="c", subcore_axis_name="s",
                                  num_cores=_NC)

    @functools.partial(
        pl.kernel,
        mesh=mesh,
        out_type=jax.ShapeDtypeStruct((E,), jnp.int32),
        compiler_params=pltpu.CompilerParams(needs_layout_passes=False),
        scratch_types=[
            pltpu.VMEM((_L,), jnp.int32),
            pltpu.VMEM((_L,), jnp.int32),
            pltpu.VMEM((_L,), jnp.int32),
            pltpu.VMEM((_L,), jnp.int32),
            pltpu.VMEM((e_per,), jnp.int32),
            pltpu.SemaphoreType.DMA,
            pltpu.SemaphoreType.DMA,
            pltpu.SemaphoreType.DMA,
            pltpu.SemaphoreType.DMA,
            pltpu.SemaphoreType.DMA,
            pltpu.SemaphoreType.DMA,
        ],
    )
    def merged_source_kernel(nrl_hbm, erl_hbm, esrc_hbm, out_hbm,
                             nrl_v, erl_v, noff_v, estart_v, src_v,
                             sem_s0, sem_s1, sem_n, sem_e, sem_o0, sem_o1):
        wid = lax.axis_index("s") * _NC + lax.axis_index("c")
        base = wid * e_per
        half = e_per // 2
        cp_s0 = pltpu.async_copy(esrc_hbm.at[pl.ds(base, half)],
                                 src_v.at[pl.ds(0, half)], sem_s0)
        cp_s1 = pltpu.async_copy(esrc_hbm.at[pl.ds(base + half, half)],
                                 src_v.at[pl.ds(half, half)], sem_s1)
        cp_n = pltpu.async_copy(nrl_hbm, nrl_v.at[pl.ds(0, B)], sem_n)
        cp_e = pltpu.async_copy(erl_hbm, erl_v.at[pl.ds(0, B)], sem_e)
        cp_n.wait()
        cp_e.wait()

        nrl = nrl_v[...]
        erl = erl_v[...]
        noff_v[...] = jnp.cumsum(nrl) - nrl
        estart_v[...] = jnp.cumsum(erl) - erl

        starts = [
            plsc.load_gather(estart_v, [jnp.full((_L,), j, jnp.int32)])
            for j in range(1, B)
        ]

        def graph_id(pos):
            gid = jnp.zeros((_L,), jnp.int32)
            for j, s in enumerate(starts):
                gid = jnp.where(pos >= s, jnp.int32(j + 1), gid)
            return gid

        gid_lo = graph_id(jnp.full((_L,), base, jnp.int32))
        gid_hi = graph_id(jnp.full((_L,), base + (e_per - 1), jnp.int32))
        uniform = jnp.min(gid_lo) == jnp.max(gid_hi)

        def writeback(h):
            return pltpu.async_copy(
                src_v.at[pl.ds(h * half, half)],
                out_hbm.at[pl.ds(base + h * half, half)],
                sem_o0 if h == 0 else sem_o1)

        @pl.when(uniform)
        def _fast():
            off = plsc.load_gather(noff_v, [gid_lo])

            cp_s0.wait()

            @plsc.parallel_loop(0, half, step=_L, unroll=8)
            def _body0(i):
                sl = pl.ds(i, _L)
                src_v[sl] = src_v[sl] + off

            cp_o0 = writeback(0)
            cp_s1.wait()

            @plsc.parallel_loop(half, e_per, step=_L, unroll=8)
            def _body1(i):
                sl = pl.ds(i, _L)
                src_v[sl] = src_v[sl] + off

            cp_o1 = writeback(1)
            cp_o0.wait()
            cp_o1.wait()

        @pl.when(jnp.logical_not(uniform))
        def _general():
            pos0 = base + lax.iota(jnp.int32, _L)

            cp_s0.wait()

            @plsc.parallel_loop(0, half, step=_L, unroll=4)
            def _body0(i):
                off = plsc.load_gather(noff_v, [graph_id(pos0 + i)])
                sl = pl.ds(i, _L)
                src_v[sl] = src_v[sl] + off

            cp_o0 = writeback(0)
            cp_s1.wait()

            @plsc.parallel_loop(half, e_per, step=_L, unroll=4)
            def _body1(i):
                off = plsc.load_gather(noff_v, [graph_id(pos0 + i)])
                sl = pl.ds(i, _L)
                src_v[sl] = src_v[sl] + off

            cp_o1 = writeback(1)
            cp_o0.wait()
            cp_o1.wait()

    return merged_source_kernel


def kernel(node_features, node_row_lengths, edge_source, edge_target,
           edge_row_lengths):
    B = node_row_lengths.shape[0]
    E = edge_source.shape[0]
    return _build(B, E)(node_row_lengths, edge_row_lengths, edge_source)
